# baseline (speedup 1.0000x reference)
import functools

import jax
import jax.numpy as jnp
from jax import lax
from jax.experimental import pallas as pl
from jax.experimental.pallas import tpu as pltpu
from jax.experimental.pallas import tpu_sc as plsc

_K = 8192
_D = 64
_N = 16384
_R = 4096
_C = 1024
_BETA = 0.25
_DP = 128


def _argmin_body(z2_ref, z_ref, e_ref, e2_ref, idx_ref):
    z = z_ref[...]
    z2 = z2_ref[...]
    best_val = jnp.full((_R, 1), jnp.inf, jnp.float32)
    best_idx = jnp.zeros((_R, 1), jnp.int32)
    for c in range(_K // _C):
        e = e_ref[pl.ds(c * _C, _C), :]
        m = lax.dot_general(z, e, (((1,), (1,)), ((), ())),
                            preferred_element_type=jnp.float32)
        dist = (z2 - 2.0 * m) + e2_ref[0:1, pl.ds(c * _C, _C)]
        dmin = jnp.min(dist, axis=1, keepdims=True)
        kio = lax.broadcasted_iota(jnp.int32, (_R, _C), 1)
        cand = jnp.min(jnp.where(dist == dmin, kio, jnp.int32(2**30)),
                       axis=1, keepdims=True) + c * _C
        better = dmin < best_val
        best_idx = jnp.where(better, cand, best_idx)
        best_val = jnp.where(better, dmin, best_val)
    idx_ref[...] = jnp.broadcast_to(best_idx, (_R, 8))


_argmin_call = pl.pallas_call(
    _argmin_body,
    grid=(_N // _R,),
    in_specs=[
        pl.BlockSpec((_R, 1), lambda i: (i, 0)),
        pl.BlockSpec((_R, _DP), lambda i: (i, 0)),
        pl.BlockSpec((_K, _DP), lambda i: (0, 0)),
        pl.BlockSpec((8, _K), lambda i: (0, 0)),
    ],
    out_specs=pl.BlockSpec((_R, 8), lambda i: (i, 0)),
    out_shape=jax.ShapeDtypeStruct((_N, 8), jnp.int32),
)

_NC = 2
_NS = 16
_NW = _NC * _NS
_BPW = _N // _NW
_GW = 128
_GCH = _BPW // _GW


def _gather_body(table_hbm, idx_hbm, out_hbm, idx_v, rows_v, sem):
    wid = lax.axis_index("s") * _NC + lax.axis_index("c")
    pltpu.sync_copy(idx_hbm.at[wid], idx_v)
    copies = [
        pltpu.async_copy(table_hbm.at[idx_v.at[c]],
                         rows_v.at[pl.ds(c * _GW, _GW)], sem)
        for c in range(_GCH)
    ]
    for cp in copies:
        cp.wait()
    pltpu.sync_copy(rows_v, out_hbm.at[pl.ds(wid * _BPW, _BPW)])


@functools.lru_cache(maxsize=None)
def _gather_call():
    return pl.kernel(
        _gather_body,
        out_type=jax.ShapeDtypeStruct((_N, _DP), jnp.float32),
        mesh=plsc.VectorSubcoreMesh(core_axis_name="c", subcore_axis_name="s",
                                    num_cores=_NC, num_subcores=_NS),
        scratch_types=[
            pltpu.VMEM((_GCH, _GW), jnp.int32),
            pltpu.VMEM((_BPW, _DP), jnp.float32),
            pltpu.SemaphoreType.DMA,
        ],
    )


def _final_body(z_ref, q_ref, st_ref, loss_ref):
    d = q_ref[:, : _D] - z_ref[...]
    st_ref[...] = z_ref[...] + d
    loss_ref[0, 0] = (1.0 + _BETA) * (jnp.sum(d * d) / jnp.float32(_N * _D))


_final_call = pl.pallas_call(
    _final_body,
    grid=(1,),
    in_specs=[
        pl.BlockSpec((_N, _D), lambda i: (0, 0)),
        pl.BlockSpec((_N, _DP), lambda i: (0, 0)),
    ],
    out_specs=[
        pl.BlockSpec((_N, _D), lambda i: (0, 0)),
        pl.BlockSpec(memory_space=pltpu.SMEM),
    ],
    out_shape=[
        jax.ShapeDtypeStruct((_N, _D), jnp.float32),
        jax.ShapeDtypeStruct((1, 1), jnp.float32),
    ],
)


def kernel(z_e, embeddings):
    B, Dd, H, W = z_e.shape
    z_flat = jnp.transpose(z_e, (0, 2, 3, 1)).reshape(-1, Dd)
    z2 = jnp.sum(z_flat ** 2, axis=1, keepdims=True)
    e2 = jnp.broadcast_to(jnp.sum(embeddings ** 2, axis=1)[None, :], (8, _K))
    z_pad = jnp.pad(z_flat, ((0, 0), (0, _DP - _D)))
    table_pad = jnp.pad(embeddings, ((0, 0), (0, _DP - _D)))
    idx = _argmin_call(z2, z_pad, table_pad, e2)[:, 0]
    z_q_pad = _gather_call()(table_pad, idx.reshape(_NW, _GCH, _GW))
    z_st_flat, loss = _final_call(z_flat, z_q_pad)
    z_q_st = jnp.transpose(z_st_flat.reshape(B, H, W, Dd), (0, 3, 1, 2))
    return z_q_st, loss[0, 0]


# speedup vs baseline: 1.2298x; 1.2298x over previous
import functools

import jax
import jax.numpy as jnp
from jax import lax
from jax.experimental import pallas as pl
from jax.experimental.pallas import tpu as pltpu
from jax.experimental.pallas import tpu_sc as plsc

_K = 8192
_D = 64
_N = 16384
_R = 2048
_C = 1024
_BETA = 0.25
_DP = 128


def _argmin_body(z2_ref, z_ref, e_ref, e2_ref, idx_ref):
    z = z_ref[...]
    z2 = z2_ref[...]
    best_val = jnp.full((_R, 1), jnp.inf, jnp.float32)
    best_idx = jnp.zeros((_R, 1), jnp.int32)
    for c in range(_K // _C):
        e = e_ref[pl.ds(c * _C, _C), :]
        m = lax.dot_general(z, e, (((1,), (1,)), ((), ())),
                            preferred_element_type=jnp.float32)
        dist = (z2 - 2.0 * m) + e2_ref[0:1, pl.ds(c * _C, _C)]
        dmin = jnp.min(dist, axis=1, keepdims=True)
        kio = lax.broadcasted_iota(jnp.int32, (_R, _C), 1)
        cand = jnp.min(jnp.where(dist == dmin, kio, jnp.int32(2**30)),
                       axis=1, keepdims=True) + c * _C
        better = dmin < best_val
        best_idx = jnp.where(better, cand, best_idx)
        best_val = jnp.where(better, dmin, best_val)
    idx_ref[...] = jnp.broadcast_to(best_idx, (_R, 8))


_argmin_call = pl.pallas_call(
    _argmin_body,
    grid=(_N // _R,),
    in_specs=[
        pl.BlockSpec((_R, 1), lambda i: (i, 0)),
        pl.BlockSpec((_R, _DP), lambda i: (i, 0)),
        pl.BlockSpec((_K, _DP), lambda i: (0, 0)),
        pl.BlockSpec((8, _K), lambda i: (0, 0)),
    ],
    out_specs=pl.BlockSpec((_R, 8), lambda i: (i, 0)),
    out_shape=jax.ShapeDtypeStruct((_N, 8), jnp.int32),
)

_NC = 2
_NS = 16
_NW = _NC * _NS
_BPW = _N // _NW
_GW = 128
_GCH = _BPW // _GW


def _gather_body(table_hbm, idx_hbm, out_hbm, idx_v, rows_v, sem):
    wid = lax.axis_index("s") * _NC + lax.axis_index("c")
    pltpu.sync_copy(idx_hbm.at[wid], idx_v)
    copies = [
        pltpu.async_copy(table_hbm.at[idx_v.at[c]],
                         rows_v.at[pl.ds(c * _GW, _GW)], sem)
        for c in range(_GCH)
    ]
    for cp in copies:
        cp.wait()
    pltpu.sync_copy(rows_v, out_hbm.at[pl.ds(wid * _BPW, _BPW)])


@functools.lru_cache(maxsize=None)
def _gather_call():
    return pl.kernel(
        _gather_body,
        out_type=jax.ShapeDtypeStruct((_N, _DP), jnp.float32),
        mesh=plsc.VectorSubcoreMesh(core_axis_name="c", subcore_axis_name="s",
                                    num_cores=_NC, num_subcores=_NS),
        scratch_types=[
            pltpu.VMEM((_GCH, _GW), jnp.int32),
            pltpu.VMEM((_BPW, _DP), jnp.float32),
            pltpu.SemaphoreType.DMA,
        ],
    )


def _final_body(z_ref, q_ref, st_ref, loss_ref):
    d = q_ref[:, : _D] - z_ref[...]
    st_ref[...] = z_ref[...] + d
    loss_ref[0, 0] = (1.0 + _BETA) * (jnp.sum(d * d) / jnp.float32(_N * _D))


_final_call = pl.pallas_call(
    _final_body,
    grid=(1,),
    in_specs=[
        pl.BlockSpec((_N, _D), lambda i: (0, 0)),
        pl.BlockSpec((_N, _DP), lambda i: (0, 0)),
    ],
    out_specs=[
        pl.BlockSpec((_N, _D), lambda i: (0, 0)),
        pl.BlockSpec(memory_space=pltpu.SMEM),
    ],
    out_shape=[
        jax.ShapeDtypeStruct((_N, _D), jnp.float32),
        jax.ShapeDtypeStruct((1, 1), jnp.float32),
    ],
)


def kernel(z_e, embeddings):
    B, Dd, H, W = z_e.shape
    z_flat = jnp.transpose(z_e, (0, 2, 3, 1)).reshape(-1, Dd)
    z2 = jnp.sum(z_flat ** 2, axis=1, keepdims=True)
    e2 = jnp.broadcast_to(jnp.sum(embeddings ** 2, axis=1)[None, :], (8, _K))
    z_pad = jnp.pad(z_flat, ((0, 0), (0, _DP - _D)))
    table_pad = jnp.pad(embeddings, ((0, 0), (0, _DP - _D)))
    idx = _argmin_call(z2, z_pad, table_pad, e2)[:, 0]
    z_q_pad = _gather_call()(table_pad, idx.reshape(_NW, _GCH, _GW))
    z_st_flat, loss = _final_call(z_flat, z_q_pad)
    z_q_st = jnp.transpose(z_st_flat.reshape(B, H, W, Dd), (0, 3, 1, 2))
    return z_q_st, loss[0, 0]
